# baseline (device time: 890564 ns/iter reference)
import jax
import jax.numpy as jnp
from jax import lax
from jax.experimental import pallas as pl
from jax.experimental.pallas import tpu as pltpu

N_DEV = 32
N_TOK = 512
D = 256
H = 512
E_LOC = 4
N_EXP = 128
XCOL = 0
ACOL = D
RCOL = D + H
WIDTH = 896


def kernel(x, router_W, route_idx, expert_W, shared_W):
    def body(x_ref, router_ref, route_ref, expert_ref, shared_ref,
             out_ref, data, send_sems, recv_sems, credit_sem):
        my = lax.axis_index("i")
        left = lax.rem(my + N_DEV - 1, N_DEV)
        right = lax.rem(my + 1, N_DEV)

        barrier = pltpu.get_barrier_semaphore()
        for nbr in (left, right):
            pl.semaphore_signal(barrier, inc=1, device_id=(nbr,),
                                device_id_type=pl.DeviceIdType.MESH)
        pl.semaphore_wait(barrier, 2)

        xv = x_ref[:, :]
        scores = jnp.dot(xv, router_ref[:, :],
                         preferred_element_type=jnp.float32)
        scores = scores - jnp.max(scores, axis=1, keepdims=True)
        p = jnp.exp(scores)
        p = p / jnp.sum(p, axis=1, keepdims=True)
        eids = lax.broadcasted_iota(jnp.int32, (N_TOK, N_EXP), 1)
        w = jnp.sum(jnp.where(eids == route_ref[:, :], p, 0.0),
                    axis=1, keepdims=True)
        data[0, :, XCOL:XCOL + D] = xv * w
        data[0, :, ACOL:ACOL + H] = jnp.dot(
            xv, shared_ref[:, :], preferred_element_type=jnp.float32)
        data[0, :, RCOL:RCOL + 1] = route_ref[:, :].astype(jnp.float32)

        ew = expert_ref[:, :, :].reshape(E_LOC * D, H)
        base = my * E_LOC

        def accumulate(s):
            xb = data[s, :, XCOL:XCOL + D]
            rt = data[s, :, RCOL:RCOL + 1]
            parts = [
                jnp.where(rt == (base + j).astype(jnp.float32), xb, 0.0)
                for j in range(E_LOC)
            ]
            xs = jnp.concatenate(parts, axis=1)
            contrib = jnp.dot(xs, ew, preferred_element_type=jnp.float32)
            data[s, :, ACOL:ACOL + H] = data[s, :, ACOL:ACOL + H] + contrib

        def send_desc(s):
            return pltpu.make_async_remote_copy(
                src_ref=data.at[s], dst_ref=data.at[1 - s],
                send_sem=send_sems.at[s], recv_sem=recv_sems.at[1 - s],
                device_id=(right,), device_id_type=pl.DeviceIdType.MESH)

        def recv_desc(s):
            return pltpu.make_async_remote_copy(
                src_ref=data.at[s], dst_ref=data.at[s],
                send_sem=send_sems.at[s], recv_sem=recv_sems.at[s],
                device_id=(left,), device_id_type=pl.DeviceIdType.MESH)

        accumulate(0)
        send_desc(0).start()

        for h in range(1, N_DEV):
            s = h % 2
            recv_desc(s).wait_recv()
            accumulate(s)
            send_desc(1 - s).wait_send()
            pl.semaphore_signal(credit_sem, inc=1, device_id=(left,),
                                device_id_type=pl.DeviceIdType.MESH)
            pl.semaphore_wait(credit_sem, 1)
            send_desc(s).start()

        send_desc(1).wait_send()
        recv_desc(0).wait_recv()
        out_ref[:, :] = data[0, :, ACOL:ACOL + H]

    return pl.pallas_call(
        body,
        out_shape=jax.ShapeDtypeStruct((N_TOK, H), jnp.float32),
        in_specs=[pl.BlockSpec(memory_space=pltpu.VMEM)] * 5,
        out_specs=pl.BlockSpec(memory_space=pltpu.VMEM),
        scratch_shapes=[
            pltpu.VMEM((2, N_TOK, WIDTH), jnp.float32),
            pltpu.SemaphoreType.DMA((2,)),
            pltpu.SemaphoreType.DMA((2,)),
            pltpu.SemaphoreType.REGULAR,
        ],
        compiler_params=pltpu.CompilerParams(collective_id=0),
    )(x, router_W, route_idx, expert_W, shared_W)


# device time: 463561 ns/iter; 1.9211x vs baseline; 1.9211x over previous
import jax
import jax.numpy as jnp
from jax import lax
from jax.experimental import pallas as pl
from jax.experimental.pallas import tpu as pltpu

N_DEV = 32
N_TOK = 512
D = 256
H = 512
E_LOC = 4
N_EXP = 128
XCOL = 0
ACOL = D
RCOL = D + H
WIDTH = 896


def kernel(x, router_W, route_idx, expert_W, shared_W):
    def body(x_ref, router_ref, route_ref, expert_ref, shared_ref,
             out_ref, data, send_sems, recv_sems, credit_sem):
        my = lax.axis_index("i")
        left = lax.rem(my + N_DEV - 1, N_DEV)
        right = lax.rem(my + 1, N_DEV)

        barrier = pltpu.get_barrier_semaphore()
        for nbr in (left, right):
            pl.semaphore_signal(barrier, inc=1, device_id=(nbr,),
                                device_id_type=pl.DeviceIdType.MESH)
        pl.semaphore_wait(barrier, 2)

        xv = x_ref[:, :]
        scores = jnp.dot(xv, router_ref[:, :],
                         preferred_element_type=jnp.float32)
        scores = scores - jnp.max(scores, axis=1, keepdims=True)
        p = jnp.exp(scores)
        p = p / jnp.sum(p, axis=1, keepdims=True)
        eids = lax.broadcasted_iota(jnp.int32, (N_TOK, N_EXP), 1)
        w = jnp.sum(jnp.where(eids == route_ref[:, :], p, 0.0),
                    axis=1, keepdims=True)
        data[0, :, XCOL:XCOL + D] = (xv * w).astype(jnp.bfloat16)
        data[0, :, ACOL:ACOL + H] = jnp.dot(
            xv, shared_ref[:, :],
            preferred_element_type=jnp.float32).astype(jnp.bfloat16)
        data[0, :, RCOL:RCOL + 1] = route_ref[:, :].astype(jnp.bfloat16)

        ew = expert_ref[:, :, :].reshape(E_LOC * D, H).astype(jnp.bfloat16)
        base = my * E_LOC

        for h in range(N_DEV):
            s = h % 2
            if h > 0:
                recv = pltpu.make_async_remote_copy(
                    src_ref=data.at[s], dst_ref=data.at[s],
                    send_sem=send_sems.at[s], recv_sem=recv_sems.at[s],
                    device_id=(left,), device_id_type=pl.DeviceIdType.MESH)
                recv.wait_recv()

            xb = data[s, :, XCOL:XCOL + D]
            rt = data[s, :, RCOL:RCOL + 1]
            parts = [
                jnp.where(rt == (base + j).astype(jnp.bfloat16), xb,
                          jnp.bfloat16(0.0))
                for j in range(E_LOC)
            ]
            xs = jnp.concatenate(parts, axis=1)
            contrib = jnp.dot(xs, ew, preferred_element_type=jnp.float32)
            acc = data[s, :, ACOL:ACOL + H].astype(jnp.float32)
            data[s, :, ACOL:ACOL + H] = (acc + contrib).astype(jnp.bfloat16)

            if h > 0:
                pl.semaphore_wait(credit_sem, 1)
            send = pltpu.make_async_remote_copy(
                src_ref=data.at[s], dst_ref=data.at[1 - s],
                send_sem=send_sems.at[s], recv_sem=recv_sems.at[1 - s],
                device_id=(right,), device_id_type=pl.DeviceIdType.MESH)
            send.start()
            send.wait_send()
            if h < N_DEV - 1:
                pl.semaphore_signal(credit_sem, inc=1, device_id=(left,),
                                    device_id_type=pl.DeviceIdType.MESH)

        fin = pltpu.make_async_remote_copy(
            src_ref=data.at[0], dst_ref=data.at[0],
            send_sem=send_sems.at[0], recv_sem=recv_sems.at[0],
            device_id=(left,), device_id_type=pl.DeviceIdType.MESH)
        fin.wait_recv()
        out_ref[:, :] = data[0, :, ACOL:ACOL + H].astype(jnp.float32)

    return pl.pallas_call(
        body,
        out_shape=jax.ShapeDtypeStruct((N_TOK, H), jnp.float32),
        in_specs=[pl.BlockSpec(memory_space=pltpu.VMEM)] * 5,
        out_specs=pl.BlockSpec(memory_space=pltpu.VMEM),
        scratch_shapes=[
            pltpu.VMEM((2, N_TOK, WIDTH), jnp.bfloat16),
            pltpu.SemaphoreType.DMA((2,)),
            pltpu.SemaphoreType.DMA((2,)),
            pltpu.SemaphoreType.REGULAR,
        ],
        compiler_params=pltpu.CompilerParams(collective_id=0),
    )(x, router_W, route_idx, expert_W, shared_W)


# device time: 348775 ns/iter; 2.5534x vs baseline; 1.3291x over previous
import jax
import jax.numpy as jnp
from jax import lax
from jax.experimental import pallas as pl
from jax.experimental.pallas import tpu as pltpu

N_DEV = 32
N_TOK = 512
D = 256
H = 512
E_LOC = 4
N_EXP = 128
RCOL = D
AWIDTH = 384


def kernel(x, router_W, route_idx, expert_W, shared_W):
    def body(x_ref, router_ref, route_ref, expert_ref, shared_ref,
             out_ref, xbuf, abuf, sendx_sems, recvx_sems,
             senda_sems, recva_sems, creditx_sem, credita_sem):
        my = lax.axis_index("i")
        left = lax.rem(my + N_DEV - 1, N_DEV)
        right = lax.rem(my + 1, N_DEV)

        barrier = pltpu.get_barrier_semaphore()
        for nbr in (left, right):
            pl.semaphore_signal(barrier, inc=1, device_id=(nbr,),
                                device_id_type=pl.DeviceIdType.MESH)
        pl.semaphore_wait(barrier, 2)

        xv = x_ref[:, :]
        scores = jnp.dot(xv, router_ref[:, :],
                         preferred_element_type=jnp.float32)
        scores = scores - jnp.max(scores, axis=1, keepdims=True)
        p = jnp.exp(scores)
        p = p / jnp.sum(p, axis=1, keepdims=True)
        eids = lax.broadcasted_iota(jnp.int32, (N_TOK, N_EXP), 1)
        w = jnp.sum(jnp.where(eids == route_ref[:, :], p, 0.0),
                    axis=1, keepdims=True)
        xbuf[0, :, 0:D] = (xv * w).astype(jnp.bfloat16)
        xbuf[0, :, RCOL:RCOL + 1] = route_ref[:, :].astype(jnp.bfloat16)
        abuf[0, :, :] = jnp.dot(
            xv, shared_ref[:, :],
            preferred_element_type=jnp.float32).astype(jnp.bfloat16)

        ew = expert_ref[:, :, :].reshape(E_LOC * D, H).astype(jnp.bfloat16)
        base = my * E_LOC

        def copy(buf, s, d, send_sems, recv_sems, dev):
            return pltpu.make_async_remote_copy(
                src_ref=buf.at[s], dst_ref=buf.at[d],
                send_sem=send_sems.at[s], recv_sem=recv_sems.at[d],
                device_id=(dev,), device_id_type=pl.DeviceIdType.MESH)

        for h in range(N_DEV):
            s = h % 2

            if h > 0:
                copy(xbuf, s, s, sendx_sems, recvx_sems, left).wait_recv()
            if 1 <= h <= N_DEV - 2:
                pl.semaphore_wait(creditx_sem, 1)
            if h <= N_DEV - 2:
                sendx = copy(xbuf, s, 1 - s, sendx_sems, recvx_sems, right)
                sendx.start()

            xb = xbuf[s, :, 0:D]
            rt = xbuf[s, :, RCOL:RCOL + 1]
            parts = [
                jnp.where(rt == (base + j).astype(jnp.bfloat16), xb,
                          jnp.bfloat16(0.0))
                for j in range(E_LOC)
            ]
            xs = jnp.concatenate(parts, axis=1)
            contrib = jnp.dot(xs, ew, preferred_element_type=jnp.float32)

            if h > 0:
                copy(abuf, s, s, senda_sems, recva_sems, left).wait_recv()
                pl.semaphore_wait(credita_sem, 1)
            acc = abuf[s, :, :].astype(jnp.float32)
            abuf[s, :, :] = (acc + contrib).astype(jnp.bfloat16)
            senda = copy(abuf, s, 1 - s, senda_sems, recva_sems, right)
            senda.start()

            if h <= N_DEV - 2:
                sendx.wait_send()
                if h <= N_DEV - 3:
                    pl.semaphore_signal(creditx_sem, inc=1, device_id=(left,),
                                        device_id_type=pl.DeviceIdType.MESH)
            senda.wait_send()
            if h <= N_DEV - 2:
                pl.semaphore_signal(credita_sem, inc=1, device_id=(left,),
                                    device_id_type=pl.DeviceIdType.MESH)

        copy(abuf, 0, 0, senda_sems, recva_sems, left).wait_recv()
        out_ref[:, :] = abuf[0, :, :].astype(jnp.float32)

    return pl.pallas_call(
        body,
        out_shape=jax.ShapeDtypeStruct((N_TOK, H), jnp.float32),
        in_specs=[pl.BlockSpec(memory_space=pltpu.VMEM)] * 5,
        out_specs=pl.BlockSpec(memory_space=pltpu.VMEM),
        scratch_shapes=[
            pltpu.VMEM((2, N_TOK, AWIDTH), jnp.bfloat16),
            pltpu.VMEM((2, N_TOK, H), jnp.bfloat16),
            pltpu.SemaphoreType.DMA((2,)),
            pltpu.SemaphoreType.DMA((2,)),
            pltpu.SemaphoreType.DMA((2,)),
            pltpu.SemaphoreType.DMA((2,)),
            pltpu.SemaphoreType.REGULAR,
            pltpu.SemaphoreType.REGULAR,
        ],
        compiler_params=pltpu.CompilerParams(collective_id=0),
    )(x, router_W, route_idx, expert_W, shared_W)


# device time: 332993 ns/iter; 2.6744x vs baseline; 1.0474x over previous
import jax
import jax.numpy as jnp
from jax import lax
from jax.experimental import pallas as pl
from jax.experimental.pallas import tpu as pltpu

N_DEV = 32
N_TOK = 512
D = 256
H = 512
E_LOC = 4
N_EXP = 128
RCOL = D
AWIDTH = 384


def kernel(x, router_W, route_idx, expert_W, shared_W):
    def body(x_ref, router_ref, route_ref, expert_ref, shared_ref,
             out_ref, xbuf, abuf, sendx_sems, recvx_sems,
             senda_sems, recva_sems, creditx_sem, credita_sem):
        my = lax.axis_index("i")
        left = lax.rem(my + N_DEV - 1, N_DEV)
        right = lax.rem(my + 1, N_DEV)

        barrier = pltpu.get_barrier_semaphore()
        for nbr in (left, right):
            pl.semaphore_signal(barrier, inc=1, device_id=(nbr,),
                                device_id_type=pl.DeviceIdType.MESH)
        pl.semaphore_wait(barrier, 2)

        xv = x_ref[:, :]
        scores = jnp.dot(xv, router_ref[:, :],
                         preferred_element_type=jnp.float32)
        scores = scores - jnp.max(scores, axis=1, keepdims=True)
        p = jnp.exp(scores)
        p = p / jnp.sum(p, axis=1, keepdims=True)
        eids = lax.broadcasted_iota(jnp.int32, (N_TOK, N_EXP), 1)
        w = jnp.sum(jnp.where(eids == route_ref[:, :], p, 0.0),
                    axis=1, keepdims=True)
        xbuf[0, :, 0:D] = (xv * w).astype(jnp.bfloat16)
        xbuf[0, :, RCOL:RCOL + 1] = route_ref[:, :].astype(jnp.bfloat16)
        abuf[0, :, :] = jnp.dot(
            xv, shared_ref[:, :],
            preferred_element_type=jnp.float32).astype(jnp.bfloat16)

        ew = expert_ref[:, :, :].reshape(E_LOC * D, H).astype(jnp.bfloat16)
        base = my * E_LOC

        def copy(buf, s, d, send_sems, recv_sems, dev):
            return pltpu.make_async_remote_copy(
                src_ref=buf.at[s], dst_ref=buf.at[d],
                send_sem=send_sems.at[s], recv_sem=recv_sems.at[d],
                device_id=(dev,), device_id_type=pl.DeviceIdType.MESH)

        for h in range(N_DEV):
            s = h % 2

            if h > 0:
                copy(xbuf, s, s, sendx_sems, recvx_sems, left).wait_recv()
            if 1 <= h <= N_DEV - 2:
                pl.semaphore_wait(creditx_sem, 1)
            if h <= N_DEV - 2:
                sendx = copy(xbuf, s, 1 - s, sendx_sems, recvx_sems, right)
                sendx.start()

            xb = xbuf[s, :, 0:D]
            rt = xbuf[s, :, RCOL:RCOL + 1]
            parts = [
                jnp.where(rt == (base + j).astype(jnp.bfloat16), xb,
                          jnp.bfloat16(0.0))
                for j in range(E_LOC)
            ]
            xs = jnp.concatenate(parts, axis=1)
            contrib = jnp.dot(xs, ew, preferred_element_type=jnp.float32)

            if h > 0:
                copy(abuf, 1 - s, s, senda_sems, recva_sems, right).wait_send()
                pl.semaphore_signal(credita_sem, inc=1, device_id=(left,),
                                    device_id_type=pl.DeviceIdType.MESH)

            if h > 0:
                copy(abuf, s, s, senda_sems, recva_sems, left).wait_recv()
                pl.semaphore_wait(credita_sem, 1)
            acc = abuf[s, :, :].astype(jnp.float32)
            abuf[s, :, :] = (acc + contrib).astype(jnp.bfloat16)
            copy(abuf, s, 1 - s, senda_sems, recva_sems, right).start()

            if h <= N_DEV - 2:
                sendx.wait_send()
                if h <= N_DEV - 3:
                    pl.semaphore_signal(creditx_sem, inc=1, device_id=(left,),
                                        device_id_type=pl.DeviceIdType.MESH)

        copy(abuf, 1, 0, senda_sems, recva_sems, right).wait_send()
        copy(abuf, 0, 0, senda_sems, recva_sems, left).wait_recv()
        out_ref[:, :] = abuf[0, :, :].astype(jnp.float32)

    return pl.pallas_call(
        body,
        out_shape=jax.ShapeDtypeStruct((N_TOK, H), jnp.float32),
        in_specs=[pl.BlockSpec(memory_space=pltpu.VMEM)] * 5,
        out_specs=pl.BlockSpec(memory_space=pltpu.VMEM),
        scratch_shapes=[
            pltpu.VMEM((2, N_TOK, AWIDTH), jnp.bfloat16),
            pltpu.VMEM((2, N_TOK, H), jnp.bfloat16),
            pltpu.SemaphoreType.DMA((2,)),
            pltpu.SemaphoreType.DMA((2,)),
            pltpu.SemaphoreType.DMA((2,)),
            pltpu.SemaphoreType.DMA((2,)),
            pltpu.SemaphoreType.REGULAR,
            pltpu.SemaphoreType.REGULAR,
        ],
        compiler_params=pltpu.CompilerParams(collective_id=0),
    )(x, router_W, route_idx, expert_W, shared_W)
